# baseline (speedup 1.0000x reference)
import math

import jax
import jax.numpy as jnp
import numpy as np
from jax.experimental import pallas as pl

_T = 2048
_PRED = 96
_K = 8
_F = 1024

_k = np.arange(1, _F + 1, dtype=np.int64)
_t = np.arange(_T, dtype=np.int64)
_ang = (2.0 * math.pi / _T) * ((_k[:, None] * _t[None, :]) % _T)
_CSM = np.concatenate(
    [np.cos(_ang), np.sin(_ang)], axis=0).astype(np.float32)


def _seasonality_kernel(x_ref, csm_ref, out_ref):
    xb = jnp.concatenate([x_ref[0], x_ref[1]], axis=1)
    csm = csm_ref[...]
    pq = jax.lax.dot_general(
        csm, xb, dimension_numbers=((((1,), (0,))), ((), ())),
        preferred_element_type=jnp.float32,
        precision=jax.lax.Precision.HIGHEST)
    p = pq[:_F]
    q = pq[_F:]
    mag = p * p + q * q
    row = jax.lax.broadcasted_iota(jnp.int32, mag.shape, 0)
    vals = jnp.where(row == _F - 1, -1.0, mag)
    sel = jnp.zeros(mag.shape, dtype=jnp.bool_)
    for _ in range(_K):
        m = jnp.max(vals, axis=0, keepdims=True)
        eligible = vals == m
        idx = jnp.min(jnp.where(eligible, row, _F), axis=0, keepdims=True)
        onehot = row == idx
        sel = jnp.logical_or(sel, onehot)
        vals = jnp.where(onehot, -2.0, vals)
    c = 2.0 / _T
    sel2 = jnp.concatenate([sel, sel], axis=0)
    ab = jnp.where(sel2, pq * c, 0.0)
    head = jax.lax.dot_general(
        csm, ab, dimension_numbers=((((0,), (0,))), ((), ())),
        preferred_element_type=jnp.float32,
        precision=jax.lax.Precision.HIGHEST)
    out_ref[0, :_T, :] = head[:, :64]
    out_ref[0, _T:, :] = head[:_PRED, :64]
    out_ref[1, :_T, :] = head[:, 64:]
    out_ref[1, _T:, :] = head[:_PRED, 64:]


def _impl(x):
    bsz, t, d = x.shape
    csm = jnp.asarray(_CSM)
    return pl.pallas_call(
        _seasonality_kernel,
        grid=(bsz // 2,),
        in_specs=[
            pl.BlockSpec((2, t, d), lambda i: (i, 0, 0)),
            pl.BlockSpec((2 * _F, _T), lambda i: (0, 0)),
        ],
        out_specs=pl.BlockSpec((2, t + _PRED, d), lambda i: (i, 0, 0)),
        out_shape=jax.ShapeDtypeStruct((bsz, t + _PRED, d), jnp.float32),
    )(x, csm)


def kernel(x):
    return _impl(x)


# speedup vs baseline: 2.1456x; 2.1456x over previous
import math

import jax
import jax.numpy as jnp
import ml_dtypes
import numpy as np
from jax.experimental import pallas as pl

_T = 2048
_PRED = 96
_K = 8
_F = 1024

_k = np.arange(1, _F + 1, dtype=np.int64)
_t = np.arange(_T, dtype=np.int64)
_ang = (2.0 * math.pi / _T) * ((_k[:, None] * _t[None, :]) % _T)
_CSM = np.concatenate([np.cos(_ang), np.sin(_ang)], axis=0)
_C0 = _CSM.astype(ml_dtypes.bfloat16)
_C1 = (_CSM - _C0.astype(np.float64)).astype(ml_dtypes.bfloat16)


def _seasonality_kernel(x_ref, c0_ref, c1_ref, out_ref):
    xb = jnp.concatenate([x_ref[0], x_ref[1]], axis=1)
    c0 = c0_ref[...]
    c1 = c1_ref[...]
    x0 = xb.astype(jnp.bfloat16)
    x1 = (xb - x0.astype(jnp.float32)).astype(jnp.bfloat16)

    def dot3(dn, r0, r1):
        acc = jax.lax.dot_general(
            c0, r0, dimension_numbers=(dn, ((), ())),
            preferred_element_type=jnp.float32)
        acc += jax.lax.dot_general(
            c0, r1, dimension_numbers=(dn, ((), ())),
            preferred_element_type=jnp.float32)
        acc += jax.lax.dot_general(
            c1, r0, dimension_numbers=(dn, ((), ())),
            preferred_element_type=jnp.float32)
        return acc

    pq = dot3(((1,), (0,)), x0, x1)
    p = pq[:_F]
    q = pq[_F:]
    mag = p * p + q * q
    row = jax.lax.broadcasted_iota(jnp.int32, mag.shape, 0)
    vals = jnp.where(row == _F - 1, -1.0, mag)
    sel = jnp.zeros(mag.shape, dtype=jnp.bool_)
    for _ in range(_K):
        m = jnp.max(vals, axis=0, keepdims=True)
        eligible = vals == m
        idx = jnp.min(jnp.where(eligible, row, _F), axis=0, keepdims=True)
        onehot = row == idx
        sel = jnp.logical_or(sel, onehot)
        vals = jnp.where(onehot, -2.0, vals)
    c = 2.0 / _T
    sel2 = jnp.concatenate([sel, sel], axis=0)
    ab = jnp.where(sel2, pq * c, 0.0)
    a0 = ab.astype(jnp.bfloat16)
    a1 = (ab - a0.astype(jnp.float32)).astype(jnp.bfloat16)
    head = dot3(((0,), (0,)), a0, a1)
    out_ref[0, :_T, :] = head[:, :64]
    out_ref[0, _T:, :] = head[:_PRED, :64]
    out_ref[1, :_T, :] = head[:, 64:]
    out_ref[1, _T:, :] = head[:_PRED, 64:]


def _impl(x):
    bsz, t, d = x.shape
    c0 = jnp.asarray(_C0)
    c1 = jnp.asarray(_C1)
    return pl.pallas_call(
        _seasonality_kernel,
        grid=(bsz // 2,),
        in_specs=[
            pl.BlockSpec((2, t, d), lambda i: (i, 0, 0)),
            pl.BlockSpec((2 * _F, _T), lambda i: (0, 0)),
            pl.BlockSpec((2 * _F, _T), lambda i: (0, 0)),
        ],
        out_specs=pl.BlockSpec((2, t + _PRED, d), lambda i: (i, 0, 0)),
        out_shape=jax.ShapeDtypeStruct((bsz, t + _PRED, d), jnp.float32),
    )(x, c0, c1)


def kernel(x):
    return _impl(x)


# speedup vs baseline: 2.6668x; 1.2429x over previous
import math

import jax
import jax.numpy as jnp
import ml_dtypes
import numpy as np
from jax.experimental import pallas as pl

_T = 2048
_PRED = 96
_K = 8
_F = 1024

_k = np.arange(1, _F + 1, dtype=np.int64)
_t = np.arange(_T, dtype=np.int64)
_ang = (2.0 * math.pi / _T) * ((_k[:, None] * _t[None, :]) % _T)
_CSM = np.concatenate([np.cos(_ang), np.sin(_ang)], axis=0)
_C0 = _CSM.astype(ml_dtypes.bfloat16)
_C1 = (_CSM - _C0.astype(np.float64)).astype(ml_dtypes.bfloat16)


def _seasonality_kernel(x_ref, c0_ref, c1_ref, out_ref):
    xb = jnp.concatenate([x_ref[0], x_ref[1]], axis=1)
    c0 = c0_ref[...]
    c1 = c1_ref[...]
    x0 = xb.astype(jnp.bfloat16)
    x1 = (xb - x0.astype(jnp.float32)).astype(jnp.bfloat16)

    def dot3(dn, r0, r1):
        acc = jax.lax.dot_general(
            c0, r0, dimension_numbers=(dn, ((), ())),
            preferred_element_type=jnp.float32)
        acc += jax.lax.dot_general(
            c0, r1, dimension_numbers=(dn, ((), ())),
            preferred_element_type=jnp.float32)
        acc += jax.lax.dot_general(
            c1, r0, dimension_numbers=(dn, ((), ())),
            preferred_element_type=jnp.float32)
        return acc

    pq = dot3(((1,), (0,)), x0, x1)
    p = pq[:_F]
    q = pq[_F:]
    mag = p * p + q * q
    row = jax.lax.broadcasted_iota(jnp.int32, mag.shape, 0)
    vals = jnp.where(row == _F - 1, -1.0, mag)
    sel = jnp.zeros(mag.shape, dtype=jnp.bool_)
    for _ in range(_K):
        m = jnp.max(vals, axis=0, keepdims=True)
        eligible = vals == m
        idx = jnp.min(jnp.where(eligible, row, _F), axis=0, keepdims=True)
        onehot = row == idx
        sel = jnp.logical_or(sel, onehot)
        vals = jnp.where(onehot, -2.0, vals)
    c = 2.0 / _T
    sel2 = jnp.concatenate([sel, sel], axis=0)
    ab = jnp.where(sel2, pq * c, 0.0)
    a0 = ab.astype(jnp.bfloat16)
    a1 = (ab - a0.astype(jnp.float32)).astype(jnp.bfloat16)
    head = jax.lax.dot_general(
        c0, a0, dimension_numbers=((((0,), (0,))), ((), ())),
        preferred_element_type=jnp.float32)
    head += jax.lax.dot_general(
        c0, a1, dimension_numbers=((((0,), (0,))), ((), ())),
        preferred_element_type=jnp.float32)
    out_ref[0, :_T, :] = head[:, :64]
    out_ref[0, _T:, :] = head[:_PRED, :64]
    out_ref[1, :_T, :] = head[:, 64:]
    out_ref[1, _T:, :] = head[:_PRED, 64:]


def _impl(x):
    bsz, t, d = x.shape
    c0 = jnp.asarray(_C0)
    c1 = jnp.asarray(_C1)
    return pl.pallas_call(
        _seasonality_kernel,
        grid=(bsz // 2,),
        in_specs=[
            pl.BlockSpec((2, t, d), lambda i: (i, 0, 0)),
            pl.BlockSpec((2 * _F, _T), lambda i: (0, 0)),
            pl.BlockSpec((2 * _F, _T), lambda i: (0, 0)),
        ],
        out_specs=pl.BlockSpec((2, t + _PRED, d), lambda i: (i, 0, 0)),
        out_shape=jax.ShapeDtypeStruct((bsz, t + _PRED, d), jnp.float32),
    )(x, c0, c1)


def kernel(x):
    return _impl(x)


# speedup vs baseline: 5.8514x; 2.1942x over previous
import math

import jax
import jax.numpy as jnp
import ml_dtypes
import numpy as np
from jax.experimental import pallas as pl

_T = 2048
_H = 1024
_PRED = 96
_K = 8

_m = np.arange(512, dtype=np.int64)
_s = np.arange(_H, dtype=np.int64)
_ang_e = (2.0 * math.pi / _H) * ((_m[:, None] * _s[None, :]) % _H)
_ang_o = (2.0 * math.pi / _T) * (((2 * _m[:, None] + 1) * _s[None, :]) % _T)


def _split(a):
    hi = a.astype(ml_dtypes.bfloat16)
    lo = (a - hi.astype(np.float64)).astype(ml_dtypes.bfloat16)
    return hi, lo


_AE = np.concatenate([np.cos(_ang_e), np.sin(_ang_e)], axis=0)
_AO = np.concatenate([np.cos(_ang_o), np.sin(_ang_o)], axis=0)
_AE0, _AE1 = _split(_AE)
_AO0, _AO1 = _split(_AO)


def _seasonality_kernel(x_ref, ae0_ref, ae1_ref, ao0_ref, ao1_ref, out_ref):
    xb = jnp.concatenate([x_ref[0], x_ref[1]], axis=1)
    ae0 = ae0_ref[...]
    ae1 = ae1_ref[...]
    ao0 = ao0_ref[...]
    ao1 = ao1_ref[...]
    e = xb[:_H] + xb[_H:]
    o = xb[:_H] - xb[_H:]

    def bsplit(a):
        hi = a.astype(jnp.bfloat16)
        lo = (a - hi.astype(jnp.float32)).astype(jnp.bfloat16)
        return hi, lo

    def mm(lhs, rhs, dn):
        return jax.lax.dot_general(
            lhs, rhs, dimension_numbers=(dn, ((), ())),
            preferred_element_type=jnp.float32)

    e0, e1 = bsplit(e)
    o0, o1 = bsplit(o)
    fwd = ((1,), (0,))
    ge = mm(ae0, e0, fwd) + mm(ae0, e1, fwd) + mm(ae1, e0, fwd)
    go = mm(ao0, o0, fwd) + mm(ao0, o1, fwd) + mm(ao1, o0, fwd)
    pe, qe = ge[:512], ge[512:]
    po, qo = go[:512], go[512:]
    mag = jnp.concatenate([pe * pe + qe * qe, po * po + qo * qo], axis=0)
    r = jax.lax.broadcasted_iota(jnp.int32, mag.shape, 0)
    kmap = jnp.where(r < 512, 2 * r, 2 * r - 1023)
    vals = jnp.where(r == 0, -1.0, mag)
    sel = jnp.zeros(mag.shape, dtype=jnp.bool_)
    for _ in range(_K):
        mx = jnp.max(vals, axis=0, keepdims=True)
        eligible = vals == mx
        idx = jnp.min(jnp.where(eligible, kmap, _T), axis=0, keepdims=True)
        onehot = kmap == idx
        sel = jnp.logical_or(sel, onehot)
        vals = jnp.where(onehot, -2.0, vals)
    c = 2.0 / _T
    sel_e = jnp.concatenate([sel[:512], sel[:512]], axis=0)
    sel_o = jnp.concatenate([sel[512:], sel[512:]], axis=0)
    abe = jnp.where(sel_e, ge * c, 0.0)
    abo = jnp.where(sel_o, go * c, 0.0)
    abe0, abe1 = bsplit(abe)
    abo0, abo1 = bsplit(abo)
    inv = ((0,), (0,))
    he = mm(ae0, abe0, inv) + mm(ae0, abe1, inv)
    ho = mm(ao0, abo0, inv) + mm(ao0, abo1, inv)
    lo_half = he + ho
    hi_half = he - ho
    out_ref[0, :_H, :] = lo_half[:, :64]
    out_ref[0, _H:_T, :] = hi_half[:, :64]
    out_ref[0, _T:, :] = lo_half[:_PRED, :64]
    out_ref[1, :_H, :] = lo_half[:, 64:]
    out_ref[1, _H:_T, :] = hi_half[:, 64:]
    out_ref[1, _T:, :] = lo_half[:_PRED, 64:]


def _impl(x):
    bsz, t, d = x.shape
    consts = [jnp.asarray(a) for a in (_AE0, _AE1, _AO0, _AO1)]
    cspec = pl.BlockSpec((_H, _H), lambda i: (0, 0))
    return pl.pallas_call(
        _seasonality_kernel,
        grid=(bsz // 2,),
        in_specs=[
            pl.BlockSpec((2, t, d), lambda i: (i, 0, 0)),
            cspec, cspec, cspec, cspec,
        ],
        out_specs=pl.BlockSpec((2, t + _PRED, d), lambda i: (i, 0, 0)),
        out_shape=jax.ShapeDtypeStruct((bsz, t + _PRED, d), jnp.float32),
    )(x, *consts)


def kernel(x):
    return _impl(x)


# speedup vs baseline: 6.1986x; 1.0593x over previous
import math

import jax
import jax.numpy as jnp
import ml_dtypes
import numpy as np
from jax.experimental import pallas as pl
from jax.experimental.pallas import tpu as pltpu

_T = 2048
_H = 1024
_PRED = 96
_K = 8

_m = np.arange(512, dtype=np.int64)
_s = np.arange(_H, dtype=np.int64)
_ang_e = (2.0 * math.pi / _H) * ((_m[:, None] * _s[None, :]) % _H)
_ang_o = (2.0 * math.pi / _T) * (((2 * _m[:, None] + 1) * _s[None, :]) % _T)


def _split(a):
    hi = a.astype(ml_dtypes.bfloat16)
    lo = (a - hi.astype(np.float64)).astype(ml_dtypes.bfloat16)
    return hi, lo


_AE = np.concatenate([np.cos(_ang_e), np.sin(_ang_e)], axis=0)
_AO = np.concatenate([np.cos(_ang_o), np.sin(_ang_o)], axis=0)
_AE0, _AE1 = _split(_AE)
_AO0, _AO1 = _split(_AO)


def _seasonality_kernel(x_ref, ae0_ref, ae1_ref, ao0_ref, ao1_ref, out_ref,
                        sel_ref):
    xb = jnp.concatenate([x_ref[0], x_ref[1]], axis=1)
    ae0 = ae0_ref[...]
    ae1 = ae1_ref[...]
    ao0 = ao0_ref[...]
    ao1 = ao1_ref[...]
    e = xb[:_H] + xb[_H:]
    o = xb[:_H] - xb[_H:]

    def bsplit(a):
        hi = a.astype(jnp.bfloat16)
        lo = (a - hi.astype(jnp.float32)).astype(jnp.bfloat16)
        return hi, lo

    def mm(lhs, rhs, dn):
        return jax.lax.dot_general(
            lhs, rhs, dimension_numbers=(dn, ((), ())),
            preferred_element_type=jnp.float32)

    e0, e1 = bsplit(e)
    o0, o1 = bsplit(o)
    fwd = ((1,), (0,))
    ge = mm(ae0, e0, fwd) + mm(ae0, e1, fwd) + mm(ae1, e0, fwd)
    go = mm(ao0, o0, fwd) + mm(ao0, o1, fwd) + mm(ao1, o0, fwd)
    pe, qe = ge[:512], ge[512:]
    po, qo = go[:512], go[512:]
    mag = jnp.concatenate([pe * pe + qe * qe, po * po + qo * qo], axis=0)
    r = jax.lax.broadcasted_iota(jnp.int32, mag.shape, 0)
    vals0 = jnp.where(r == 0, -1.0, mag)
    v = vals0
    for _ in range(_K):
        mx = jnp.max(v, axis=0, keepdims=True)
        v = jnp.where(v == mx, -2.0, v)
    sel_fast = v == -2.0
    cnt = jnp.sum(jnp.where(sel_fast, 1.0, 0.0), axis=0, keepdims=True)
    sel_ref[...] = jnp.where(sel_fast, 1.0, 0.0)
    ok = jnp.max(cnt) == float(_K)

    @pl.when(jnp.logical_not(ok))
    def _exact_topk():
        kmap = jnp.where(r < 512, 2 * r, 2 * r - 1023)
        vals = vals0
        sel = jnp.zeros(mag.shape, dtype=jnp.bool_)
        for _ in range(_K):
            mx = jnp.max(vals, axis=0, keepdims=True)
            eligible = vals == mx
            idx = jnp.min(jnp.where(eligible, kmap, _T), axis=0,
                          keepdims=True)
            onehot = kmap == idx
            sel = jnp.logical_or(sel, onehot)
            vals = jnp.where(onehot, -2.0, vals)
        sel_ref[...] = jnp.where(sel, 1.0, 0.0)

    sel = sel_ref[...] == 1.0
    c = 2.0 / _T
    sel_e = jnp.concatenate([sel[:512], sel[:512]], axis=0)
    sel_o = jnp.concatenate([sel[512:], sel[512:]], axis=0)
    abe = jnp.where(sel_e, ge * c, 0.0)
    abo = jnp.where(sel_o, go * c, 0.0)
    abe0, abe1 = bsplit(abe)
    abo0, abo1 = bsplit(abo)
    inv = ((0,), (0,))
    he = mm(ae0, abe0, inv) + mm(ae0, abe1, inv)
    ho = mm(ao0, abo0, inv) + mm(ao0, abo1, inv)
    lo_half = he + ho
    hi_half = he - ho
    out_ref[0, :_H, :] = lo_half[:, :64]
    out_ref[0, _H:_T, :] = hi_half[:, :64]
    out_ref[0, _T:, :] = lo_half[:_PRED, :64]
    out_ref[1, :_H, :] = lo_half[:, 64:]
    out_ref[1, _H:_T, :] = hi_half[:, 64:]
    out_ref[1, _T:, :] = lo_half[:_PRED, 64:]


def _impl(x):
    bsz, t, d = x.shape
    consts = [jnp.asarray(a) for a in (_AE0, _AE1, _AO0, _AO1)]
    cspec = pl.BlockSpec((_H, _H), lambda i: (0, 0))
    return pl.pallas_call(
        _seasonality_kernel,
        grid=(bsz // 2,),
        in_specs=[
            pl.BlockSpec((2, t, d), lambda i: (i, 0, 0)),
            cspec, cspec, cspec, cspec,
        ],
        out_specs=pl.BlockSpec((2, t + _PRED, d), lambda i: (i, 0, 0)),
        out_shape=jax.ShapeDtypeStruct((bsz, t + _PRED, d), jnp.float32),
        scratch_shapes=[pltpu.VMEM((2 * 512, 2 * d), jnp.float32)],
    )(x, *consts)


def kernel(x):
    return _impl(x)
